# baseline (speedup 1.0000x reference)
import functools

import jax
import jax.numpy as jnp
from jax import lax
from jax.experimental import pallas as pl
from jax.experimental.pallas import tpu as pltpu
from jax.experimental.pallas import tpu_sc as plsc

UNITS = 128
DICT = 256
NC = 2
NS = 16
NW = NC * NS
L = 16
WPR = UNITS // 2

B, T = 1024, 200
N = B * T
N_SC = 122880
N_TC = N - N_SC
PER_W = N_SC // NW
CHUNK = 320
NCHUNK = PER_W // CHUNK
GROUPS = CHUNK // L
TOKT = 512

_ILV = plsc.PackFormat.INTERLEAVED


def _body(x_hbm, w_hbm, out_hbm, wp_v, dp_v, xin, out_v, insems, outsems):
    wid = lax.axis_index("s") * NC + lax.axis_index("c")
    base = wid * PER_W

    lane = lax.iota(jnp.int32, L)

    def in_copy(k, p):
        return pltpu.make_async_copy(
            x_hbm.at[pl.ds(base + k * CHUNK, CHUNK)], xin[p], insems[p]
        )

    def out_copy(k, p):
        return pltpu.make_async_copy(
            out_v[p], out_hbm.at[pl.ds(base + k * CHUNK, CHUNK)], outsems[p]
        )

    in_copy(0, 0).start()
    in_copy(1, 1).start()

    pltpu.sync_copy(w_hbm, out_v[0].at[pl.ds(0, DICT)])
    stage = out_v[0]

    @plsc.parallel_loop(0, DICT)
    def build_row(r):
        r1 = jnp.minimum(r + 1, DICT - 1)
        for q in range(WPR // L):
            lo0 = stage[r, pl.ds(q * L, L)]
            hi0 = stage[r, pl.ds(WPR + q * L, L)]
            lo1 = stage[r1, pl.ds(q * L, L)]
            hi1 = stage[r1, pl.ds(WPR + q * L, L)]
            wp_v[pl.ds(r * WPR + q * L, L)] = plsc.bitcast(
                plsc.pack(lo0, hi0, format=_ILV), jnp.int32
            )
            dp_v[pl.ds(r * WPR + q * L, L)] = plsc.bitcast(
                plsc.pack(lo1 - lo0, hi1 - hi0, format=_ILV), jnp.int32
            )

    def chunk_pair(q, _):
        for p in (0, 1):
            k = 2 * q + p
            in_copy(k, p).wait()

            @pl.when(q >= 1)
            def _wait_out():
                out_copy(k, p).wait()

            @plsc.parallel_loop(0, GROUPS)
            def group_body(g):
                xv = xin[p][pl.ds(g * L, L)]
                i0v = xv.astype(jnp.int32)
                afv = xv - i0v.astype(jnp.float32)
                a0v = i0v * WPR

                @plsc.parallel_loop(0, L, unroll=8)
                def token_body(j):
                    jv = lax.broadcast_in_dim(j, (L,), ())
                    av = jnp.take_along_axis(afv, jv, axis=0)
                    avbf = plsc.pack(av, av, format=_ILV)
                    b0 = jnp.take_along_axis(a0v, jv, axis=0) + lane
                    t = g * L + j
                    ews = [
                        plsc.load_gather(
                            wp_v.at[pl.ds(c * L, DICT * WPR - c * L)], [b0]
                        )
                        for c in range(WPR // L)
                    ]
                    dws = [
                        plsc.load_gather(
                            dp_v.at[pl.ds(c * L, DICT * WPR - c * L)], [b0]
                        )
                        for c in range(WPR // L)
                    ]
                    for c in range(WPR // L):
                        ebf = plsc.bitcast(ews[c], jnp.bfloat16)
                        dbf = plsc.bitcast(dws[c], jnp.bfloat16)
                        olo, ohi = plsc.unpack(ebf + avbf * dbf, format=_ILV)
                        out_v[p][t, pl.ds(c * L, L)] = olo
                        out_v[p][t, pl.ds(WPR + c * L, L)] = ohi

            out_copy(k, p).start()

            @pl.when(q < NCHUNK // 2 - 1)
            def _prefetch():
                in_copy(k + 2, p).start()
        return _

    lax.fori_loop(0, NCHUNK // 2, chunk_pair, None)
    out_copy(NCHUNK - 2, 0).wait()
    out_copy(NCHUNK - 1, 1).wait()


def _tc_body(x_ref, w_ref, out_ref):
    x = x_ref[0, 0]
    d = lax.broadcasted_iota(jnp.int32, (TOKT, DICT), 1).astype(jnp.float32)
    coef = jnp.maximum(1.0 - jnp.abs(d - x[:, None]), 0.0).astype(jnp.bfloat16)
    out_ref[0] = jnp.dot(coef, w_ref[...], preferred_element_type=jnp.float32)


def _run_tc(x_tc, w_bf, n_tok=N_TC):
    grid = n_tok // TOKT
    return pl.pallas_call(
        _tc_body,
        out_shape=jax.ShapeDtypeStruct((grid, TOKT, UNITS), jnp.float32),
        grid=(grid,),
        in_specs=[
            pl.BlockSpec((1, 1, TOKT), lambda i: (i, 0, 0)),
            pl.BlockSpec((DICT, UNITS), lambda i: (0, 0)),
        ],
        out_specs=pl.BlockSpec((1, TOKT, UNITS), lambda i: (i, 0, 0)),
    )(x_tc.reshape(grid, 1, TOKT), w_bf)


@jax.jit
def _run(x_flat, w):
    return _run_tc(x_flat, w.astype(jnp.bfloat16), N).reshape(N, UNITS)
    mesh = plsc.VectorSubcoreMesh(
        core_axis_name="c", subcore_axis_name="s", num_cores=NC, num_subcores=NS
    )
    out_sc = pl.kernel(
        _body,
        out_type=jax.ShapeDtypeStruct((N_SC, UNITS), jnp.float32),
        mesh=mesh,
        compiler_params=pltpu.CompilerParams(needs_layout_passes=False),
        scratch_types=[
            pltpu.VMEM((DICT * WPR,), jnp.int32),
            pltpu.VMEM((DICT * WPR,), jnp.int32),
            [pltpu.VMEM((CHUNK,), jnp.float32) for _ in range(2)],
            [pltpu.VMEM((CHUNK, UNITS), jnp.float32) for _ in range(2)],
            [pltpu.SemaphoreType.DMA for _ in range(2)],
            [pltpu.SemaphoreType.DMA for _ in range(2)],
        ],
    )(x_flat[:N_SC], w)
    out_tc = _run_tc(x_flat[N_SC:], w.astype(jnp.bfloat16))
    return jnp.concatenate([out_sc, out_tc.reshape(N_TC, UNITS)], axis=0)


def kernel(inputs, w):
    x_flat = inputs.reshape(N)
    out = _run(x_flat, w)
    return out.reshape(B, T, 1, UNITS)


# speedup vs baseline: 3.1601x; 3.1601x over previous
import functools

import jax
import jax.numpy as jnp
from jax import lax
from jax.experimental import pallas as pl
from jax.experimental.pallas import tpu as pltpu
from jax.experimental.pallas import tpu_sc as plsc

UNITS = 128
DICT = 256
NC = 2
NS = 16
NW = NC * NS
L = 16
WPR = UNITS // 2

B, T = 1024, 200
N = B * T
PER_W = N // NW
CHUNK = 160
NCHUNK = PER_W // CHUNK
GROUPS = CHUNK // L
NBUF = 4

_ILV = plsc.PackFormat.INTERLEAVED


def _body(x_hbm, w_hbm, out_hbm, wp_v, dp_v, xin, out_v, insems, outsems):
    wid = lax.axis_index("s") * NC + lax.axis_index("c")
    base = wid * PER_W

    lane = lax.iota(jnp.int32, L)

    def in_copy(k, p):
        return pltpu.make_async_copy(
            x_hbm.at[pl.ds(base + k * CHUNK, CHUNK)], xin[p], insems[p]
        )

    def out_copy(k, p):
        return pltpu.make_async_copy(
            out_v[p], out_hbm.at[pl.ds(base + k * CHUNK, CHUNK)], outsems[p]
        )

    for p in range(NBUF):
        in_copy(p, p).start()

    HALF = CHUNK
    BASE1 = 152
    pltpu.sync_copy(w_hbm.at[pl.ds(0, HALF)], out_v[0])
    pltpu.sync_copy(
        w_hbm.at[pl.ds(BASE1, DICT - BASE1)],
        out_v[1].at[pl.ds(0, DICT - BASE1)],
    )

    def build_row(stage, r, rl, r1l):
        for q in range(WPR // L):
            lo0 = stage[rl, pl.ds(q * L, L)]
            hi0 = stage[rl, pl.ds(WPR + q * L, L)]
            lo1 = stage[r1l, pl.ds(q * L, L)]
            hi1 = stage[r1l, pl.ds(WPR + q * L, L)]
            wp_v[pl.ds(r * WPR + q * L, L)] = plsc.bitcast(
                plsc.pack(lo0, hi0, format=_ILV), jnp.int32
            )
            dp_v[pl.ds(r * WPR + q * L, L)] = plsc.bitcast(
                plsc.pack(lo1 - lo0, hi1 - hi0, format=_ILV), jnp.int32
            )

    @plsc.parallel_loop(0, HALF - 1)
    def build_lo(r):
        build_row(out_v[0], r, r, r + 1)

    @plsc.parallel_loop(HALF - 1, DICT)
    def build_hi(r):
        rl = r - BASE1
        build_row(out_v[1], r, rl, jnp.minimum(rl + 1, DICT - 1 - BASE1))

    def chunk_quad(q, _):
        for p in range(NBUF):
            k = NBUF * q + p
            in_copy(k, p).wait()

            @pl.when(q >= 1)
            def _wait_out():
                out_copy(k, p).wait()

            @plsc.parallel_loop(0, GROUPS)
            def group_body(g):
                xv = xin[p][pl.ds(g * L, L)]
                i0v = xv.astype(jnp.int32)
                afv = xv - i0v.astype(jnp.float32)
                a0v = i0v * WPR

                @plsc.parallel_loop(0, L, unroll=8)
                def token_body(j):
                    jv = lax.broadcast_in_dim(j, (L,), ())
                    av = jnp.take_along_axis(afv, jv, axis=0)
                    avbf = plsc.pack(av, av, format=_ILV)
                    b0 = jnp.take_along_axis(a0v, jv, axis=0) + lane
                    t = g * L + j
                    ews = [
                        plsc.load_gather(
                            wp_v.at[pl.ds(c * L, DICT * WPR - c * L)], [b0]
                        )
                        for c in range(WPR // L)
                    ]
                    dws = [
                        plsc.load_gather(
                            dp_v.at[pl.ds(c * L, DICT * WPR - c * L)], [b0]
                        )
                        for c in range(WPR // L)
                    ]
                    for c in range(WPR // L):
                        ebf = plsc.bitcast(ews[c], jnp.bfloat16)
                        dbf = plsc.bitcast(dws[c], jnp.bfloat16)
                        olo, ohi = plsc.unpack(ebf + avbf * dbf, format=_ILV)
                        out_v[p][t, pl.ds(c * L, L)] = olo
                        out_v[p][t, pl.ds(WPR + c * L, L)] = ohi

            out_copy(k, p).start()

            @pl.when(q < NCHUNK // NBUF - 1)
            def _prefetch():
                in_copy(k + NBUF, p).start()
        return _

    lax.fori_loop(0, NCHUNK // NBUF, chunk_quad, None)
    for p in range(NBUF):
        out_copy(NCHUNK - NBUF + p, p).wait()


@jax.jit
def _run(x_flat, w):
    mesh = plsc.VectorSubcoreMesh(
        core_axis_name="c", subcore_axis_name="s", num_cores=NC, num_subcores=NS
    )
    return pl.kernel(
        _body,
        out_type=jax.ShapeDtypeStruct((N, UNITS), jnp.float32),
        mesh=mesh,
        compiler_params=pltpu.CompilerParams(needs_layout_passes=False),
        scratch_types=[
            pltpu.VMEM((DICT * WPR,), jnp.int32),
            pltpu.VMEM((DICT * WPR,), jnp.int32),
            [pltpu.VMEM((CHUNK,), jnp.float32) for _ in range(NBUF)],
            [pltpu.VMEM((CHUNK, UNITS), jnp.float32) for _ in range(NBUF)],
            [pltpu.SemaphoreType.DMA for _ in range(NBUF)],
            [pltpu.SemaphoreType.DMA for _ in range(NBUF)],
        ],
    )(x_flat, w)


def kernel(inputs, w):
    x_flat = inputs.reshape(N)
    out = _run(x_flat, w)
    return out.reshape(B, T, 1, UNITS)


# speedup vs baseline: 3.1827x; 1.0071x over previous
import functools

import jax
import jax.numpy as jnp
from jax import lax
from jax.experimental import pallas as pl
from jax.experimental.pallas import tpu as pltpu
from jax.experimental.pallas import tpu_sc as plsc

UNITS = 128
DICT = 256
NC = 2
NS = 16
NW = NC * NS
L = 16
WPR = UNITS // 2

B, T = 1024, 200
N = B * T
PER_W = N // NW
CHUNK = 160
NCHUNK = PER_W // CHUNK
GROUPS = CHUNK // L
NBUF = 4

_ILV = plsc.PackFormat.INTERLEAVED


def _body(x_hbm, w_hbm, out_hbm, wp_v, dp_v, xin, out_v, insems, outsems):
    wid = lax.axis_index("s") * NC + lax.axis_index("c")
    base = wid * PER_W

    lane = lax.iota(jnp.int32, L)

    def in_copy(k, p):
        return pltpu.make_async_copy(
            x_hbm.at[pl.ds(base + k * CHUNK, CHUNK)], xin[p], insems[p]
        )

    def out_copy(k, p):
        return pltpu.make_async_copy(
            out_v[p], out_hbm.at[pl.ds(base + k * CHUNK, CHUNK)], outsems[p]
        )

    for p in range(NBUF):
        in_copy(p, p).start()

    HALF = CHUNK
    BASE1 = 152
    stage_lo = pltpu.make_async_copy(
        w_hbm.at[pl.ds(0, HALF)], out_v[0], outsems[0]
    )
    stage_hi = pltpu.make_async_copy(
        w_hbm.at[pl.ds(BASE1, DICT - BASE1)],
        out_v[1].at[pl.ds(0, DICT - BASE1)],
        outsems[1],
    )
    stage_lo.start()
    stage_hi.start()
    stage_lo.wait()
    stage_hi.wait()

    def build_row(stage, r, rl, r1l):
        for q in range(WPR // L):
            lo0 = stage[rl, pl.ds(q * L, L)]
            hi0 = stage[rl, pl.ds(WPR + q * L, L)]
            lo1 = stage[r1l, pl.ds(q * L, L)]
            hi1 = stage[r1l, pl.ds(WPR + q * L, L)]
            wp_v[pl.ds(r * WPR + q * L, L)] = plsc.bitcast(
                plsc.pack(lo0, hi0, format=_ILV), jnp.int32
            )
            dp_v[pl.ds(r * WPR + q * L, L)] = plsc.bitcast(
                plsc.pack(lo1 - lo0, hi1 - hi0, format=_ILV), jnp.int32
            )

    @plsc.parallel_loop(0, HALF - 1)
    def build_lo(r):
        build_row(out_v[0], r, r, r + 1)

    @plsc.parallel_loop(HALF - 1, DICT)
    def build_hi(r):
        rl = r - BASE1
        build_row(out_v[1], r, rl, jnp.minimum(rl + 1, DICT - 1 - BASE1))

    def chunk_quad(q, _):
        for p in range(NBUF):
            k = NBUF * q + p
            in_copy(k, p).wait()

            @pl.when(q >= 1)
            def _wait_out():
                out_copy(k, p).wait()

            @plsc.parallel_loop(0, GROUPS)
            def group_body(g):
                xv = xin[p][pl.ds(g * L, L)]
                i0v = xv.astype(jnp.int32)
                afv = xv - i0v.astype(jnp.float32)
                a0v = i0v * WPR

                @plsc.parallel_loop(0, L, unroll=8)
                def token_body(j):
                    jv = lax.broadcast_in_dim(j, (L,), ())
                    av = jnp.take_along_axis(afv, jv, axis=0)
                    avbf = plsc.pack(av, av, format=_ILV)
                    b0 = jnp.take_along_axis(a0v, jv, axis=0) + lane
                    t = g * L + j
                    ews = [
                        plsc.load_gather(
                            wp_v.at[pl.ds(c * L, DICT * WPR - c * L)], [b0]
                        )
                        for c in range(WPR // L)
                    ]
                    dws = [
                        plsc.load_gather(
                            dp_v.at[pl.ds(c * L, DICT * WPR - c * L)], [b0]
                        )
                        for c in range(WPR // L)
                    ]
                    for c in range(WPR // L):
                        ebf = plsc.bitcast(ews[c], jnp.bfloat16)
                        dbf = plsc.bitcast(dws[c], jnp.bfloat16)
                        olo, ohi = plsc.unpack(ebf + avbf * dbf, format=_ILV)
                        out_v[p][t, pl.ds(c * L, L)] = olo
                        out_v[p][t, pl.ds(WPR + c * L, L)] = ohi

            out_copy(k, p).start()

            @pl.when(q < NCHUNK // NBUF - 1)
            def _prefetch():
                in_copy(k + NBUF, p).start()
        return _

    lax.fori_loop(0, NCHUNK // NBUF, chunk_quad, None)
    for p in range(NBUF):
        out_copy(NCHUNK - NBUF + p, p).wait()


@jax.jit
def _run(x_flat, w):
    mesh = plsc.VectorSubcoreMesh(
        core_axis_name="c", subcore_axis_name="s", num_cores=NC, num_subcores=NS
    )
    return pl.kernel(
        _body,
        out_type=jax.ShapeDtypeStruct((N, UNITS), jnp.float32),
        mesh=mesh,
        compiler_params=pltpu.CompilerParams(needs_layout_passes=False),
        scratch_types=[
            pltpu.VMEM((DICT * WPR,), jnp.int32),
            pltpu.VMEM((DICT * WPR,), jnp.int32),
            [pltpu.VMEM((CHUNK,), jnp.float32) for _ in range(NBUF)],
            [pltpu.VMEM((CHUNK, UNITS), jnp.float32) for _ in range(NBUF)],
            [pltpu.SemaphoreType.DMA for _ in range(NBUF)],
            [pltpu.SemaphoreType.DMA for _ in range(NBUF)],
        ],
    )(x_flat, w)


def kernel(inputs, w):
    x_flat = inputs.reshape(N)
    out = _run(x_flat, w)
    return out.reshape(B, T, 1, UNITS)


# speedup vs baseline: 3.1896x; 1.0022x over previous
import jax
import jax.numpy as jnp
from jax import lax
from jax.experimental import pallas as pl
from jax.experimental.pallas import tpu as pltpu
from jax.experimental.pallas import tpu_sc as plsc

UNITS = 128
DICT = 256
NC = 2
NS = 16
NW = NC * NS
L = 16
WPR = UNITS // 2

B, T = 1024, 200
N = B * T
PER_W = N // NW
CHUNK = 160
NCHUNK = PER_W // CHUNK
GROUPS = CHUNK // L
NBUF = 4

_ILV = plsc.PackFormat.INTERLEAVED


def _body(x_hbm, w_hbm, out_hbm, wp_v, dp_v, xin, out_v, insems, outsems):
    wid = lax.axis_index("s") * NC + lax.axis_index("c")
    base = wid * PER_W

    lane = lax.iota(jnp.int32, L)

    def in_copy(k, p):
        return pltpu.make_async_copy(
            x_hbm.at[pl.ds(base + k * CHUNK, CHUNK)], xin[p], insems[p]
        )

    def out_copy(k, p):
        return pltpu.make_async_copy(
            out_v[p], out_hbm.at[pl.ds(base + k * CHUNK, CHUNK)], outsems[p]
        )

    for p in range(NBUF):
        in_copy(p, p).start()

    HALF = CHUNK
    BASE1 = 152
    stage_lo = pltpu.make_async_copy(
        w_hbm.at[pl.ds(0, HALF)], out_v[0], outsems[0]
    )
    stage_hi = pltpu.make_async_copy(
        w_hbm.at[pl.ds(BASE1, DICT - BASE1)],
        out_v[1].at[pl.ds(0, DICT - BASE1)],
        outsems[1],
    )
    stage_lo.start()
    stage_hi.start()
    stage_lo.wait()
    stage_hi.wait()

    def build_row(stage, r, rl, r1l):
        for q in range(WPR // L):
            lo0 = stage[rl, pl.ds(q * L, L)]
            hi0 = stage[rl, pl.ds(WPR + q * L, L)]
            lo1 = stage[r1l, pl.ds(q * L, L)]
            hi1 = stage[r1l, pl.ds(WPR + q * L, L)]
            wp_v[pl.ds(r * WPR + q * L, L)] = plsc.bitcast(
                plsc.pack(lo0, hi0, format=_ILV), jnp.int32
            )
            dp_v[pl.ds(r * WPR + q * L, L)] = plsc.bitcast(
                plsc.pack(lo1 - lo0, hi1 - hi0, format=_ILV), jnp.int32
            )

    @plsc.parallel_loop(0, HALF - 1)
    def build_lo(r):
        build_row(out_v[0], r, r, r + 1)

    @plsc.parallel_loop(HALF - 1, DICT)
    def build_hi(r):
        rl = r - BASE1
        build_row(out_v[1], r, rl, jnp.minimum(rl + 1, DICT - 1 - BASE1))

    def chunk_quad(q, _):
        for p in range(NBUF):
            k = NBUF * q + p
            in_copy(k, p).wait()

            @pl.when(q >= 1)
            def _wait_out():
                out_copy(k, p).wait()

            @plsc.parallel_loop(0, GROUPS)
            def group_body(g):
                xv = xin[p][pl.ds(g * L, L)]
                i0v = xv.astype(jnp.int32)
                afv = xv - i0v.astype(jnp.float32)
                a0v = i0v * WPR

                @plsc.parallel_loop(0, L, unroll=8)
                def token_body(j):
                    jv = lax.broadcast_in_dim(j, (L,), ())
                    av = jnp.take_along_axis(afv, jv, axis=0)
                    avbf = plsc.pack(av, av, format=_ILV)
                    b0 = jnp.take_along_axis(a0v, jv, axis=0) + lane
                    t = g * L + j
                    ews = [
                        plsc.load_gather(
                            wp_v.at[pl.ds(c * L, DICT * WPR - c * L)], [b0]
                        )
                        for c in range(WPR // L)
                    ]
                    dws = [
                        plsc.load_gather(
                            dp_v.at[pl.ds(c * L, DICT * WPR - c * L)], [b0]
                        )
                        for c in range(WPR // L)
                    ]
                    for c in range(WPR // L):
                        ebf = plsc.bitcast(ews[c], jnp.bfloat16)
                        dbf = plsc.bitcast(dws[c], jnp.bfloat16)
                        olo, ohi = plsc.unpack(ebf + avbf * dbf, format=_ILV)
                        out_v[p][t, pl.ds(c * L, L)] = olo
                        out_v[p][t, pl.ds(WPR + c * L, L)] = ohi

            out_copy(k, p).start()

            @pl.when(q < NCHUNK // NBUF - 1)
            def _prefetch():
                in_copy(k + NBUF, p).start()
        return _

    lax.fori_loop(0, NCHUNK // NBUF, chunk_quad, None)
    for p in range(NBUF):
        out_copy(NCHUNK - NBUF + p, p).wait()


@jax.jit
def _run(x_flat, w):
    mesh = plsc.VectorSubcoreMesh(
        core_axis_name="c", subcore_axis_name="s", num_cores=NC, num_subcores=NS
    )
    return pl.kernel(
        _body,
        out_type=jax.ShapeDtypeStruct((N, UNITS), jnp.float32),
        mesh=mesh,
        compiler_params=pltpu.CompilerParams(needs_layout_passes=False),
        scratch_types=[
            pltpu.VMEM((DICT * WPR,), jnp.int32),
            pltpu.VMEM((DICT * WPR,), jnp.int32),
            [pltpu.VMEM((CHUNK,), jnp.float32) for _ in range(NBUF)],
            [pltpu.VMEM((CHUNK, UNITS), jnp.float32) for _ in range(NBUF)],
            [pltpu.SemaphoreType.DMA for _ in range(NBUF)],
            [pltpu.SemaphoreType.DMA for _ in range(NBUF)],
        ],
    )(x_flat, w)


def kernel(inputs, w):
    x_flat = inputs.reshape(N)
    out = _run(x_flat, w)
    return out.reshape(B, T, 1, UNITS)
